# Initial kernel scaffold; baseline (speedup 1.0000x reference)
import jax
import jax.numpy as jnp
from jax.experimental import pallas as pl


def kernel(x, edge_index, W_pool0, b_pool0, W_neigh0, W_self0, b_self0, W_pool1, b_pool1, W_neigh1, W_self1, b_self1, W_pool2, b_pool2, W_neigh2, W_self2, b_self2):
    raise NotImplementedError("write your pallas kernel here")


# speedup vs baseline: 1.8740x; 1.8740x over previous
import functools

import jax
import jax.numpy as jnp
from jax import lax
from jax.experimental import pallas as pl
from jax.experimental.pallas import tpu as pltpu
from jax.experimental.pallas import tpu_sc as plsc

_N = 10000
_E = 320000
_D = 128

_NC = 2
_NS = 16
_NW = _NC * _NS
_ROWS = 320
_PADN = _NW * _ROWS
_CHUNK = 6400
_NCHUNKS = _E // _CHUNK
_GROUPS = _CHUNK // 16
_CAPM = 16384

_SC_PARAMS = pltpu.CompilerParams(needs_layout_passes=False)
_MESH = plsc.VectorSubcoreMesh(core_axis_name="c", subcore_axis_name="s")


def _worker():
    return lax.axis_index("s") * _NC + lax.axis_index("c")


def _bin_body(src_hbm, dst_hbm, mrows_hbm, cnt_hbm,
              srcb_v, dstb_v, mpack_v, tmp16_v, sem):
    wid = _worker()
    lo = wid * _ROWS

    def chunk_body(ci, m):
        pltpu.sync_copy(src_hbm.at[pl.ds(ci * _CHUNK, _CHUNK)], srcb_v)
        pltpu.sync_copy(dst_hbm.at[pl.ds(ci * _CHUNK, _CHUNK)], dstb_v)

        def scan_group(g, m):
            dvec = dstb_v[pl.ds(g * 16, 16)]
            svec = srcb_v[pl.ds(g * 16, 16)]
            lov = jnp.full((16,), lo, jnp.int32)
            msk = (dvec >= lov) & (dvec < lov + _ROWS)
            pos = plsc.cumsum(msk.astype(jnp.int32))
            idx = m + pos - 1
            packed = svec * 512 + (dvec - lov)
            plsc.store_scatter(mpack_v, [idx], packed, mask=msk)
            return m + pos[15]

        return lax.fori_loop(0, _GROUPS, scan_group, m)

    m = lax.fori_loop(0, _NCHUNKS, chunk_body, 0)

    mpack_v[pl.ds(m, 16)] = jnp.full((16,), _ROWS, jnp.int32)

    pltpu.sync_copy(mpack_v, mrows_hbm.at[wid])
    tmp16_v[pl.ds(0, 16)] = jnp.full((16,), m, jnp.int32)
    pltpu.sync_copy(tmp16_v, cnt_hbm.at[wid])


_bin = functools.partial(
    pl.kernel,
    _bin_body,
    out_type=(jax.ShapeDtypeStruct((_NW, _CAPM + 16), jnp.int32),
              jax.ShapeDtypeStruct((_NW, 16), jnp.int32)),
    mesh=_MESH,
    scratch_types=[
        pltpu.VMEM((_CHUNK,), jnp.int32),
        pltpu.VMEM((_CHUNK,), jnp.int32),
        pltpu.VMEM((_CAPM + 16,), jnp.int32),
        pltpu.VMEM((16,), jnp.int32),
        pltpu.SemaphoreType.DMA,
    ],
    compiler_params=_SC_PARAMS,
)()


def _layer_body(hp_hbm, mrows_hbm, cnt_hbm, out_hbm,
                acc_v, mpack_v, sidx_v, cnt16_v, rows_v, sem):
    wid = _worker()
    lo = wid * _ROWS
    zero_f = jnp.zeros((16,), jnp.float32)

    def zero_body(i, carry):
        for k in range(8):
            acc_v[i, pl.ds(k * 16, 16)] = zero_f
        return carry

    lax.fori_loop(0, _ROWS + 1, zero_body, 0)

    pltpu.sync_copy(cnt_hbm.at[wid], cnt16_v)
    pltpu.sync_copy(mrows_hbm.at[wid], mpack_v)
    m = cnt16_v[pl.ds(0, 16)][0]
    nb = (m + 15) // 16

    def gb(j, carry):
        pv = mpack_v[pl.ds(j * 16, 16)]
        sidx_v[pl.ds(0, 16)] = pv >> 9
        dv = pv & 511
        pltpu.async_copy(hp_hbm.at[sidx_v], rows_v, sem).wait()
        for e in range(16):
            d = dv[e]
            for k in range(8):
                sl = pl.ds(k * 16, 16)
                acc_v[d, sl] = jnp.maximum(acc_v[d, sl], rows_v[e, sl])
        return carry

    lax.fori_loop(0, nb, gb, 0)

    pltpu.sync_copy(acc_v.at[pl.ds(0, _ROWS)], out_hbm.at[pl.ds(lo, _ROWS)])


_layer = functools.partial(
    pl.kernel,
    _layer_body,
    out_type=jax.ShapeDtypeStruct((_PADN, _D), jnp.float32),
    mesh=_MESH,
    scratch_types=[
        pltpu.VMEM((_ROWS + 1, _D), jnp.float32),
        pltpu.VMEM((_CAPM + 16,), jnp.int32),
        pltpu.VMEM((16,), jnp.int32),
        pltpu.VMEM((16,), jnp.int32),
        pltpu.VMEM((16, _D), jnp.float32),
        pltpu.SemaphoreType.DMA,
    ],
    compiler_params=_SC_PARAMS,
)()


_RB = 1000
_GRID = _N // _RB


def _pre_body(x_ref, wp_ref, bp_ref, hp_ref):
    hp_ref[...] = jnp.maximum(
        jnp.dot(x_ref[...], wp_ref[...], preferred_element_type=jnp.float32)
        + bp_ref[...], 0.0)


def _mid_body(h_ref, agg_ref, ws_ref, bs_ref, wn_ref, wp_ref, bp_ref,
              h2_ref, hp2_ref):
    t = (jnp.dot(h_ref[...], ws_ref[...], preferred_element_type=jnp.float32)
         + bs_ref[...]
         + jnp.dot(agg_ref[...], wn_ref[...], preferred_element_type=jnp.float32))
    h2 = jnp.where(t > 0.0, t, jnp.exp(jnp.minimum(t, 0.0)) - 1.0)
    h2_ref[...] = h2
    hp2_ref[...] = jnp.maximum(
        jnp.dot(h2, wp_ref[...], preferred_element_type=jnp.float32)
        + bp_ref[...], 0.0)


def _fin_body(h_ref, agg_ref, ws_ref, bs_ref, wn_ref, out_ref):
    out_ref[...] = (
        jnp.dot(h_ref[...], ws_ref[...], preferred_element_type=jnp.float32)
        + bs_ref[...]
        + jnp.dot(agg_ref[...], wn_ref[...], preferred_element_type=jnp.float32))


_row_spec = pl.BlockSpec((_RB, _D), lambda i: (i, 0))
_w_spec = pl.BlockSpec((_D, _D), lambda i: (0, 0))
_b_spec = pl.BlockSpec((1, _D), lambda i: (0, 0))
_f32 = jnp.float32

_pre = pl.pallas_call(
    _pre_body,
    grid=(_GRID,),
    in_specs=[_row_spec, _w_spec, _b_spec],
    out_specs=_row_spec,
    out_shape=jax.ShapeDtypeStruct((_N, _D), _f32),
)

_mid = pl.pallas_call(
    _mid_body,
    grid=(_GRID,),
    in_specs=[_row_spec, _row_spec, _w_spec, _b_spec, _w_spec, _w_spec, _b_spec],
    out_specs=[_row_spec, _row_spec],
    out_shape=[jax.ShapeDtypeStruct((_N, _D), _f32),
               jax.ShapeDtypeStruct((_N, _D), _f32)],
)

_fin = pl.pallas_call(
    _fin_body,
    grid=(_GRID,),
    in_specs=[_row_spec, _row_spec, _w_spec, _b_spec, _w_spec],
    out_specs=_row_spec,
    out_shape=jax.ShapeDtypeStruct((_N, _D), _f32),
)


def kernel(x, edge_index,
           W_pool0, b_pool0, W_neigh0, W_self0, b_self0,
           W_pool1, b_pool1, W_neigh1, W_self1, b_self1,
           W_pool2, b_pool2, W_neigh2, W_self2, b_self2):
    src = edge_index[0]
    dst = edge_index[1]
    bp0 = b_pool0.reshape(1, _D)
    bp1 = b_pool1.reshape(1, _D)
    bp2 = b_pool2.reshape(1, _D)
    bs0 = b_self0.reshape(1, _D)
    bs1 = b_self1.reshape(1, _D)
    bs2 = b_self2.reshape(1, _D)

    mrows, cnts = _bin(src, dst)

    hp0 = _pre(x, W_pool0, bp0)
    agg0 = _layer(hp0, mrows, cnts)[:_N]
    h1, hp1 = _mid(x, agg0, W_self0, bs0, W_neigh0, W_pool1, bp1)
    agg1 = _layer(hp1, mrows, cnts)[:_N]
    h2, hp2 = _mid(h1, agg1, W_self1, bs1, W_neigh1, W_pool2, bp2)
    agg2 = _layer(hp2, mrows, cnts)[:_N]
    return _fin(h2, agg2, W_self2, bs2, W_neigh2)
